# baseline (speedup 1.0000x reference)
import functools

import jax
import jax.numpy as jnp
from jax import lax
from jax.experimental import pallas as pl
from jax.experimental.pallas import tpu as pltpu

_S = 8
_K = 4


def _perm_kernel(gamma_ref, gammat_ref, noise_ref, noiset_ref, out_ref,
                 buf_ref, sem_ref, *, n, ngrid):
    i = pl.program_id(0)
    nq = n // 8

    gamma_row = gamma_ref[...]
    gamma_col = gammat_ref[...]
    ones_row = jnp.ones((1, n), dtype=jnp.float32)
    iota8 = lax.broadcasted_iota(jnp.int32, (8, n), 0)

    for k in range(_K):
        @pl.when(i > 0)
        def _wait_prev(k=k):
            pltpu.make_async_copy(
                buf_ref.at[k],
                out_ref.at[pl.ds(((i - 1) * _K + k) * _S, _S)],
                sem_ref.at[k],
            ).wait()

        rank_total = jnp.zeros((), jnp.float32)
        for t in range(_S):
            ts = k * _S + t
            row = gamma_row + noise_ref[ts]
            col = gamma_col + noiset_ref[ts]
            cnt = jnp.where(col < row, 1.0, 0.0)
            rank = lax.dot(ones_row, cnt)
            rank_total = rank_total + jnp.sum(rank)
            rank_i = rank.astype(jnp.int32)
            rank_q = lax.shift_right_logical(rank_i, 3)
            rank_m = jnp.bitwise_and(rank_i, 7)
            sub_hit = iota8 == rank_m
            for q in range(nq):
                hit = sub_hit & (rank_q == q)
                buf_ref[k, t, 8 * q:8 * q + 8, :] = jnp.where(hit, 1.0, 0.0)

        @pl.when(rank_total != float(_S * (n * (n - 1) // 2)))
        def _patch(k=k):
            kx = lax.broadcasted_iota(jnp.int32, (n, n), 0)
            jx = lax.broadcasted_iota(jnp.int32, (n, n), 1)
            kxf = kx.astype(jnp.float32)
            trilf = jnp.where(kx < jx, 1.0, 0.0)
            for t in range(_S):
                ts = k * _S + t
                row = gamma_row + noise_ref[ts]
                col = gamma_col + noiset_ref[ts]
                le = col <= row
                eq = col == row
                cnt = jnp.where(le, jnp.where(eq, trilf, 1.0), 0.0)
                rank = lax.dot(ones_row, cnt)
                buf_ref[k, t] = (kxf == rank).astype(jnp.float32)

        pltpu.make_async_copy(
            buf_ref.at[k],
            out_ref.at[pl.ds((i * _K + k) * _S, _S)],
            sem_ref.at[k],
        ).start()

    @pl.when(i == ngrid - 1)
    def _drain():
        for k in range(_K):
            pltpu.make_async_copy(
                buf_ref.at[k],
                out_ref.at[pl.ds(0, _S)],
                sem_ref.at[k],
            ).wait()


def kernel(num_samples, gamma, gumbel_noise):
    n = gamma.shape[0]
    s = gumbel_noise.shape[0]
    gamma2d = gamma.reshape(1, n)
    gammat = gamma.reshape(n, 1)
    noise3d = gumbel_noise.reshape(s, 1, n)
    noiset3d = gumbel_noise.reshape(s, n, 1)
    ngrid = s // (_S * _K)

    return pl.pallas_call(
        functools.partial(_perm_kernel, n=n, ngrid=ngrid),
        grid=(ngrid,),
        in_specs=[
            pl.BlockSpec((1, n), lambda i: (0, 0)),
            pl.BlockSpec((n, 1), lambda i: (0, 0)),
            pl.BlockSpec((_S * _K, 1, n), lambda i: (i, 0, 0)),
            pl.BlockSpec((_S * _K, n, 1), lambda i: (i, 0, 0)),
        ],
        out_specs=pl.BlockSpec(memory_space=pltpu.MemorySpace.HBM),
        out_shape=jax.ShapeDtypeStruct((s, n, n), jnp.float32),
        scratch_shapes=[
            pltpu.VMEM((_K, _S, n, n), jnp.float32),
            pltpu.SemaphoreType.DMA((_K,)),
        ],
        compiler_params=pltpu.CompilerParams(
            dimension_semantics=("arbitrary",),
        ),
    )(gamma2d, gammat, noise3d, noiset3d)


# speedup vs baseline: 1.5812x; 1.5812x over previous
import functools

import jax
import jax.numpy as jnp
from jax import lax
from jax.experimental import pallas as pl
from jax.experimental.pallas import tpu as pltpu

_S = 8
_K = 4


def _perm_kernel(gamma_ref, noise_ref, out_ref, buf_ref, sem_ref, *, n, ngrid):
    i = pl.program_id(0)
    nq = n // 8
    sk = _S * _K

    gamma_row = gamma_ref[...]
    scores = gamma_row + noise_ref[...].reshape(sk, n)
    scores_t = scores.T
    ones_row = jnp.ones((1, n), dtype=jnp.float32)
    iota8 = lax.broadcasted_iota(jnp.int32, (8, n), 0)

    for k in range(_K):
        @pl.when(i > 0)
        def _wait_prev(k=k):
            pltpu.make_async_copy(
                buf_ref.at[k],
                out_ref.at[pl.ds(((i - 1) * _K + k) * _S, _S)],
                sem_ref.at[k],
            ).wait()

        rank_total = jnp.zeros((), jnp.float32)
        for t in range(_S):
            ts = k * _S + t
            row = scores[ts:ts + 1, :]
            col = scores_t[:, ts:ts + 1]
            cnt = jnp.where(col < row, 1.0, 0.0)
            rank = lax.dot(ones_row, cnt)
            rank_total = rank_total + jnp.sum(rank)
            rank_i = rank.astype(jnp.int32)
            rank_q = lax.shift_right_logical(rank_i, 3)
            rank_m = jnp.bitwise_and(rank_i, 7)
            sub_hit = iota8 == rank_m
            for q in range(nq):
                hit = sub_hit & (rank_q == q)
                buf_ref[k, t, 8 * q:8 * q + 8, :] = jnp.where(hit, 1.0, 0.0)

        @pl.when(rank_total != float(_S * (n * (n - 1) // 2)))
        def _patch(k=k):
            kx = lax.broadcasted_iota(jnp.int32, (n, n), 0)
            jx = lax.broadcasted_iota(jnp.int32, (n, n), 1)
            kxf = kx.astype(jnp.float32)
            trilf = jnp.where(kx < jx, 1.0, 0.0)
            for t in range(_S):
                ts = k * _S + t
                row = scores[ts:ts + 1, :]
                col = scores_t[:, ts:ts + 1]
                le = col <= row
                eq = col == row
                cnt = jnp.where(le, jnp.where(eq, trilf, 1.0), 0.0)
                rank = lax.dot(ones_row, cnt)
                buf_ref[k, t] = (kxf == rank).astype(jnp.float32)

        pltpu.make_async_copy(
            buf_ref.at[k],
            out_ref.at[pl.ds((i * _K + k) * _S, _S)],
            sem_ref.at[k],
        ).start()

    @pl.when(i == ngrid - 1)
    def _drain():
        for k in range(_K):
            pltpu.make_async_copy(
                buf_ref.at[k],
                out_ref.at[pl.ds(0, _S)],
                sem_ref.at[k],
            ).wait()


def kernel(num_samples, gamma, gumbel_noise):
    n = gamma.shape[0]
    s = gumbel_noise.shape[0]
    gamma2d = gamma.reshape(1, n)
    noise3d = gumbel_noise.reshape(s, 1, n)
    ngrid = s // (_S * _K)

    return pl.pallas_call(
        functools.partial(_perm_kernel, n=n, ngrid=ngrid),
        grid=(ngrid,),
        in_specs=[
            pl.BlockSpec((1, n), lambda i: (0, 0)),
            pl.BlockSpec((_S * _K, 1, n), lambda i: (i, 0, 0)),
        ],
        out_specs=pl.BlockSpec(memory_space=pltpu.MemorySpace.HBM),
        out_shape=jax.ShapeDtypeStruct((s, n, n), jnp.float32),
        scratch_shapes=[
            pltpu.VMEM((_K, _S, n, n), jnp.float32),
            pltpu.SemaphoreType.DMA((_K,)),
        ],
        compiler_params=pltpu.CompilerParams(
            dimension_semantics=("arbitrary",),
        ),
    )(gamma2d, noise3d)


# speedup vs baseline: 1.6087x; 1.0174x over previous
import functools

import jax
import jax.numpy as jnp
from jax import lax
from jax.experimental import pallas as pl
from jax.experimental.pallas import tpu as pltpu

_S = 4
_K = 8


def _perm_kernel(gamma_ref, noise_ref, out_ref, buf_ref, sem_ref, *, n, ngrid):
    i = pl.program_id(0)
    nq = n // 8
    sk = _S * _K

    gamma_row = gamma_ref[...]
    scores = gamma_row + noise_ref[...].reshape(sk, n)
    scores_t = scores.T
    ones_row = jnp.ones((1, n), dtype=jnp.float32)
    iota8 = lax.broadcasted_iota(jnp.int32, (8, n), 0)

    for k in range(_K):
        @pl.when(i > 0)
        def _wait_prev(k=k):
            pltpu.make_async_copy(
                buf_ref.at[k],
                out_ref.at[pl.ds(((i - 1) * _K + k) * _S, _S)],
                sem_ref.at[k],
            ).wait()

        rank_total = jnp.zeros((), jnp.float32)
        for t in range(_S):
            ts = k * _S + t
            row = scores[ts:ts + 1, :]
            col = scores_t[:, ts:ts + 1]
            cnt = jnp.where(col < row, 1.0, 0.0)
            rank = lax.dot(ones_row, cnt)
            rank_total = rank_total + jnp.sum(rank)
            rank_i = rank.astype(jnp.int32)
            rank_q = lax.shift_right_logical(rank_i, 3)
            rank_m = jnp.bitwise_and(rank_i, 7)
            sub_hit = iota8 == rank_m
            for q in range(nq):
                hit = sub_hit & (rank_q == q)
                buf_ref[k, t, 8 * q:8 * q + 8, :] = jnp.where(hit, 1.0, 0.0)

        @pl.when(rank_total != float(_S * (n * (n - 1) // 2)))
        def _patch(k=k):
            kx = lax.broadcasted_iota(jnp.int32, (n, n), 0)
            jx = lax.broadcasted_iota(jnp.int32, (n, n), 1)
            kxf = kx.astype(jnp.float32)
            trilf = jnp.where(kx < jx, 1.0, 0.0)
            for t in range(_S):
                ts = k * _S + t
                row = scores[ts:ts + 1, :]
                col = scores_t[:, ts:ts + 1]
                le = col <= row
                eq = col == row
                cnt = jnp.where(le, jnp.where(eq, trilf, 1.0), 0.0)
                rank = lax.dot(ones_row, cnt)
                buf_ref[k, t] = (kxf == rank).astype(jnp.float32)

        pltpu.make_async_copy(
            buf_ref.at[k],
            out_ref.at[pl.ds((i * _K + k) * _S, _S)],
            sem_ref.at[k],
        ).start()

    @pl.when(i == ngrid - 1)
    def _drain():
        for k in range(_K):
            pltpu.make_async_copy(
                buf_ref.at[k],
                out_ref.at[pl.ds(0, _S)],
                sem_ref.at[k],
            ).wait()


def kernel(num_samples, gamma, gumbel_noise):
    n = gamma.shape[0]
    s = gumbel_noise.shape[0]
    gamma2d = gamma.reshape(1, n)
    noise3d = gumbel_noise.reshape(s, 1, n)
    ngrid = s // (_S * _K)

    return pl.pallas_call(
        functools.partial(_perm_kernel, n=n, ngrid=ngrid),
        grid=(ngrid,),
        in_specs=[
            pl.BlockSpec((1, n), lambda i: (0, 0)),
            pl.BlockSpec((_S * _K, 1, n), lambda i: (i, 0, 0)),
        ],
        out_specs=pl.BlockSpec(memory_space=pltpu.MemorySpace.HBM),
        out_shape=jax.ShapeDtypeStruct((s, n, n), jnp.float32),
        scratch_shapes=[
            pltpu.VMEM((_K, _S, n, n), jnp.float32),
            pltpu.SemaphoreType.DMA((_K,)),
        ],
        compiler_params=pltpu.CompilerParams(
            dimension_semantics=("arbitrary",),
        ),
    )(gamma2d, noise3d)
